# baseline (speedup 1.0000x reference)
import jax
import jax.numpy as jnp
from jax import lax
from jax.experimental import pallas as pl
from jax.experimental.pallas import tpu as pltpu
from jax.experimental.pallas import tpu_sc as plsc

B, N, D, M = 4, 4096, 1024, 1024
NC, NS = 2, 16
NW = NC * NS
RW = M // NW
T = 32
DC = D // 16


def _sc_body(emb_hbm, ids_hbm, out_hbm,
             ids_v, acc_v, rba, rbb, bnd_v, scale_v, sema, semb, semo):
    ZV = jnp.zeros((16,), jnp.float32)
    c = lax.axis_index("c")
    s = lax.axis_index("s")
    w = s * NC + c
    m0 = w * RW

    def zr(r, _):
        for jc in range(DC):
            acc_v[r, pl.ds(jc * 16, 16)] = ZV
        return 0

    lax.fori_loop(0, RW, zr, 0)

    for b in range(B):
        pltpu.sync_copy(ids_hbm.at[b], ids_v.at[pl.ds(0, N)])
        if b > 0:
            pltpu.make_async_copy(
                rba, out_hbm.at[b - 1, pl.ds(pl.multiple_of(m0, 8), RW)],
                semo).wait()

        for g in range(3):
            tgt = jax.lax.broadcasted_iota(jnp.int32, (16,), 0) + (
                m0 + g * 16)
            lo0 = jnp.zeros((16,), jnp.int32)
            hi0 = jnp.full((16,), N, jnp.int32)

            def sbody(it, carry):
                lo, hi = carry
                mid = (lo + hi) >> 1
                midc = jnp.minimum(mid, N - 1)
                v = plsc.load_gather(ids_v, [midc])
                less = jnp.logical_and(v < tgt, mid < N)
                return (jnp.where(less, mid + 1, lo),
                        jnp.where(less, hi, mid))

            lo, _hi = lax.fori_loop(0, 13, sbody, (lo0, hi0))
            bnd_v[pl.ds(g * 16, 16)] = lo

        for g in range(2):
            cnt = (bnd_v[pl.ds(g * 16 + 1, 16)] -
                   bnd_v[pl.ds(g * 16, 16)]).astype(jnp.float32)
            scale_v[pl.ds(g * 16, 16)] = 1.0 / (cnt + 1.0)

        tst = bnd_v[pl.ds(0, 16)][0]
        ten = bnd_v[pl.ds(RW, 16)][0]
        abase = (tst // 8) * 8
        nch = (ten - abase + T - 1) // T
        npair = (nch + 1) // 2

        def chunk_base(j):
            return pl.multiple_of(
                jnp.minimum(abase + j * T, N - T), 8)

        def issue(j, buf, sem):
            return pltpu.async_copy(
                emb_hbm.at[b, pl.ds(chunk_base(j), T)], buf, sem)

        def process(j, buf):
            base0 = abase + j * T
            base = chunk_base(j)
            lo_t = jnp.maximum(base0, tst)

            def row_body(r, _):
                t = base + r
                idv = ids_v[pl.ds(t, 16)][0]
                valid = jnp.logical_and(t >= lo_t, t < ten)
                mt = jnp.where(valid, idv - m0, RW)
                for jc in range(DC):
                    sl = pl.ds(jc * 16, 16)
                    plsc.addupdate(acc_v.at[mt, sl], buf[r, sl])
                return 0

            lax.fori_loop(0, T, row_body, 0)

        def ch_body(ci, _):
            pltpu.sync_copy(emb_hbm.at[b, pl.ds(chunk_base(ci), T)], rba)
            process(ci, rba)
            return 0

        lax.fori_loop(0, nch, ch_body, 0)

        def fr(r, _):
            sc = plsc.load_gather(scale_v, [jnp.full((16,), r, jnp.int32)])
            for jc in range(DC):
                sl = pl.ds(jc * 16, 16)
                rba[r, sl] = acc_v[r, sl] * sc
                acc_v[r, sl] = ZV
            return 0

        lax.fori_loop(0, RW, fr, 0)
        pltpu.async_copy(
            rba, out_hbm.at[b, pl.ds(pl.multiple_of(m0, 8), RW)], semo)

    pltpu.make_async_copy(
        rba, out_hbm.at[B - 1, pl.ds(pl.multiple_of(m0, 8), RW)],
        semo).wait()


@jax.jit
def _sc_pool(embeddings, position_ids):
    mesh = plsc.VectorSubcoreMesh(
        core_axis_name="c", subcore_axis_name="s",
        num_cores=NC, num_subcores=NS)
    return pl.kernel(
        _sc_body,
        out_type=jax.ShapeDtypeStruct((B, M, D), jnp.float32),
        mesh=mesh,
        compiler_params=pltpu.CompilerParams(needs_layout_passes=False),
        scratch_types=[
            pltpu.VMEM((N + 16,), jnp.int32),
            pltpu.VMEM((RW + 1, D), jnp.float32),
            pltpu.VMEM((T, D), jnp.float32),
            pltpu.VMEM((T, D), jnp.float32),
            pltpu.VMEM((48,), jnp.int32),
            pltpu.VMEM((RW,), jnp.float32),
            pltpu.SemaphoreType.DMA,
            pltpu.SemaphoreType.DMA,
            pltpu.SemaphoreType.DMA,
        ],
    )(embeddings, position_ids)


def kernel(embeddings, position_ids):
    return _sc_pool(embeddings, position_ids)


# speedup vs baseline: 1.0085x; 1.0085x over previous
import jax
import jax.numpy as jnp
from jax import lax
from jax.experimental import pallas as pl
from jax.experimental.pallas import tpu as pltpu
from jax.experimental.pallas import tpu_sc as plsc

B, N, D, M = 4, 4096, 1024, 1024
NC, NS = 2, 16
NW = NC * NS
RW = M // NW
T = 32
DC = D // 16


def _sc_body(emb_hbm, ids_hbm, out_hbm,
             ids_v, acc_v, rba, rbb, bnd_v, scale_v, sema, semb, semo):
    ZV = jnp.zeros((16,), jnp.float32)
    c = lax.axis_index("c")
    s = lax.axis_index("s")
    w = s * NC + c
    m0 = w * RW

    def zr(r, _):
        for jc in range(DC):
            acc_v[r, pl.ds(jc * 16, 16)] = ZV
        return 0

    lax.fori_loop(0, RW, zr, 0)

    for b in range(B):
        pltpu.sync_copy(ids_hbm.at[b], ids_v.at[pl.ds(0, N)])
        if b > 0:
            pltpu.make_async_copy(
                rba, out_hbm.at[b - 1, pl.ds(pl.multiple_of(m0, 8), RW)],
                semo).wait()

        for g in range(3):
            tgt = jax.lax.broadcasted_iota(jnp.int32, (16,), 0) + (
                m0 + g * 16)
            lo0 = jnp.zeros((16,), jnp.int32)
            hi0 = jnp.full((16,), N, jnp.int32)

            def sbody(it, carry):
                lo, hi = carry
                mid = (lo + hi) >> 1
                midc = jnp.minimum(mid, N - 1)
                v = plsc.load_gather(ids_v, [midc])
                less = jnp.logical_and(v < tgt, mid < N)
                return (jnp.where(less, mid + 1, lo),
                        jnp.where(less, hi, mid))

            lo, _hi = lax.fori_loop(0, 13, sbody, (lo0, hi0))
            bnd_v[pl.ds(g * 16, 16)] = lo

        for g in range(2):
            cnt = (bnd_v[pl.ds(g * 16 + 1, 16)] -
                   bnd_v[pl.ds(g * 16, 16)]).astype(jnp.float32)
            scale_v[pl.ds(g * 16, 16)] = 1.0 / (cnt + 1.0)

        tst = bnd_v[pl.ds(0, 16)][0]
        ten = bnd_v[pl.ds(RW, 16)][0]
        abase = (tst // 8) * 8
        nch = (ten - abase + T - 1) // T
        npair = (nch + 1) // 2

        def chunk_base(j):
            return pl.multiple_of(
                jnp.minimum(abase + j * T, N - T), 8)

        def issue(j, buf, sem):
            return pltpu.async_copy(
                emb_hbm.at[b, pl.ds(chunk_base(j), T)], buf, sem)

        def process(j, buf):
            base0 = abase + j * T
            base = chunk_base(j)
            lo_t = jnp.maximum(base0, tst)

            def row_body(r, _):
                t = base + r
                idv = ids_v[pl.ds(t, 16)][0]
                valid = jnp.logical_and(t >= lo_t, t < ten)
                mt = jnp.where(valid, idv - m0, RW)
                for jc in range(DC):
                    sl = pl.ds(jc * 16, 16)
                    plsc.addupdate(acc_v.at[mt, sl], buf[r, sl])
                return 0

            lax.fori_loop(0, T, row_body, 0)

        issue(0, rba, sema)

        def pair(ci2, _):
            j0 = 2 * ci2
            issue(j0 + 1, rbb, semb)
            pltpu.make_async_copy(emb_hbm.at[b, pl.ds(0, T)], rba,
                                  sema).wait()
            process(j0, rba)
            issue(j0 + 2, rba, sema)
            pltpu.make_async_copy(emb_hbm.at[b, pl.ds(0, T)], rbb,
                                  semb).wait()
            process(j0 + 1, rbb)
            return 0

        lax.fori_loop(0, npair, pair, 0)
        pltpu.make_async_copy(emb_hbm.at[b, pl.ds(0, T)], rba, sema).wait()

        def fr(r, _):
            sc = plsc.load_gather(scale_v, [jnp.full((16,), r, jnp.int32)])
            for jc in range(DC):
                sl = pl.ds(jc * 16, 16)
                rba[r, sl] = acc_v[r, sl] * sc
                acc_v[r, sl] = ZV
            return 0

        lax.fori_loop(0, RW, fr, 0)
        pltpu.async_copy(
            rba, out_hbm.at[b, pl.ds(pl.multiple_of(m0, 8), RW)], semo)

    pltpu.make_async_copy(
        rba, out_hbm.at[B - 1, pl.ds(pl.multiple_of(m0, 8), RW)],
        semo).wait()


@jax.jit
def _sc_pool(embeddings, position_ids):
    mesh = plsc.VectorSubcoreMesh(
        core_axis_name="c", subcore_axis_name="s",
        num_cores=NC, num_subcores=NS)
    return pl.kernel(
        _sc_body,
        out_type=jax.ShapeDtypeStruct((B, M, D), jnp.float32),
        mesh=mesh,
        compiler_params=pltpu.CompilerParams(needs_layout_passes=False),
        scratch_types=[
            pltpu.VMEM((N + 16,), jnp.int32),
            pltpu.VMEM((RW + 1, D), jnp.float32),
            pltpu.VMEM((T, D), jnp.float32),
            pltpu.VMEM((T, D), jnp.float32),
            pltpu.VMEM((48,), jnp.int32),
            pltpu.VMEM((RW,), jnp.float32),
            pltpu.SemaphoreType.DMA,
            pltpu.SemaphoreType.DMA,
            pltpu.SemaphoreType.DMA,
        ],
    )(embeddings, position_ids)


def kernel(embeddings, position_ids):
    return _sc_pool(embeddings, position_ids)


# speedup vs baseline: 4.1408x; 4.1058x over previous
import functools

import jax
import jax.numpy as jnp
from jax.experimental import pallas as pl
from jax.experimental.pallas import tpu as pltpu

B, N, D, M = 4, 4096, 1024, 1024
CK = 512
NK = N // CK


def _body(ids_ref, e_ref, o_ref, cnt_ref):
    k = pl.program_id(1)

    @pl.when(k == 0)
    def _():
        o_ref[...] = jnp.zeros_like(o_ref)
        cnt_ref[...] = jnp.zeros_like(cnt_ref)

    ids = ids_ref[0, :, pl.ds(k * CK, CK)]
    mask = (jax.lax.broadcasted_iota(jnp.int32, (M, CK), 0) == ids).astype(
        jnp.float32
    )
    o_ref[0] += jnp.dot(
        mask.astype(jnp.bfloat16), e_ref[0].astype(jnp.bfloat16),
        preferred_element_type=jnp.float32)
    cnt_ref[...] += jnp.broadcast_to(
        jnp.sum(mask, axis=1, keepdims=True), (M, 128)
    )

    @pl.when(k == NK - 1)
    def _():
        o_ref[0] = o_ref[0] / (cnt_ref[:, 0:1] + 1.0)


def kernel(embeddings, position_ids):
    ids3 = position_ids.reshape(B, 1, N)
    out = pl.pallas_call(
        _body,
        grid=(B, NK),
        in_specs=[
            pl.BlockSpec((1, 1, N), lambda b, k: (b, 0, 0)),
            pl.BlockSpec((1, CK, D), lambda b, k: (b, k, 0)),
        ],
        out_specs=pl.BlockSpec((1, M, D), lambda b, k: (b, 0, 0)),
        out_shape=jax.ShapeDtypeStruct((B, M, D), jnp.float32),
        scratch_shapes=[pltpu.VMEM((M, 128), jnp.float32)],
    )(ids3, embeddings)
    return out
